# Initial kernel scaffold; baseline (speedup 1.0000x reference)
import jax
import jax.numpy as jnp
from jax.experimental import pallas as pl


def kernel(x, batch, W1, b1, W2, b2):
    raise NotImplementedError("write your pallas kernel here")


# speedup vs baseline: 11.7028x; 11.7028x over previous
import functools

import jax
import jax.numpy as jnp
from jax import lax
from jax.experimental import pallas as pl
from jax.experimental.pallas import tpu as pltpu

B = 512


def _body(nb, nblk, x_ref, batch_ref, w1_ref, b1_ref, w2_ref, b2_ref,
          z_ref, acc_ref, dacc_ref, m_ref):
    i = pl.program_id(0)

    @pl.when(i == 0)
    def _init():
        acc_ref[...] = jnp.zeros_like(acc_ref)
        dacc_ref[...] = jnp.zeros_like(dacc_ref)
        m_ref[...] = jnp.full_like(m_ref, -jnp.inf)

    xb = x_ref[...]
    h = jnp.tanh(
        jax.lax.dot_general(xb, w1_ref[...], (((1,), (0,)), ((), ())),
                            preferred_element_type=jnp.float32)
        + b1_ref[...])
    lg = jax.lax.dot_general(w2_ref[...], h, (((1,), (1,)), ((), ())),
                             preferred_element_type=jnp.float32)
    lg = lg + b2_ref[...]

    m_old = m_ref[...]
    m_blk = jnp.max(lg, axis=(0, 1), keepdims=True)
    m_new = jnp.maximum(m_old, m_blk)
    m_ref[...] = m_new
    scale = jnp.exp(m_old - m_new)

    w = jnp.exp(lg - m_new)
    seg = lax.broadcasted_iota(jnp.int32, (B, nb), 0)
    onehot = (seg == batch_ref[0]).astype(jnp.float32)
    ew = onehot * w

    acc_ref[...] = acc_ref[...] * scale + jax.lax.dot_general(
        ew, xb, (((1,), (0,)), ((), ())), preferred_element_type=jnp.float32)
    dacc_ref[...] = dacc_ref[...] * scale + jnp.sum(ew, axis=1, keepdims=True)

    @pl.when(i == nblk - 1)
    def _finish():
        z_ref[...] = acc_ref[...] / (dacc_ref[...] + 1e-8)


@jax.jit
def kernel(x, batch, W1, b1, W2, b2):
    n, d = x.shape
    h = W1.shape[1]
    nb = 2000 if n % 2000 == 0 else n
    nblk = n // nb

    batch3 = batch.astype(jnp.int32).reshape(nblk, 1, nb)
    b1r = b1.reshape(1, h)
    w2r = W2.reshape(1, h)
    b2r = b2.reshape(1, 1)

    z = pl.pallas_call(
        functools.partial(_body, nb, nblk),
        grid=(nblk,),
        in_specs=[
            pl.BlockSpec((nb, d), lambda i: (i, 0)),
            pl.BlockSpec((1, 1, nb), lambda i: (i, 0, 0)),
            pl.BlockSpec((d, h), lambda i: (0, 0)),
            pl.BlockSpec((1, h), lambda i: (0, 0)),
            pl.BlockSpec((1, h), lambda i: (0, 0)),
            pl.BlockSpec((1, 1), lambda i: (0, 0)),
        ],
        out_specs=pl.BlockSpec((B, d), lambda i: (0, 0)),
        out_shape=jax.ShapeDtypeStruct((B, d), jnp.float32),
        scratch_shapes=[
            pltpu.VMEM((B, d), jnp.float32),
            pltpu.VMEM((B, 1), jnp.float32),
            pltpu.VMEM((1, 1), jnp.float32),
        ],
        compiler_params=pltpu.CompilerParams(
            dimension_semantics=("arbitrary",)),
    )(x, batch3, W1, b1r, w2r, b2r)
    return z
